# baseline (speedup 1.0000x reference)
import functools

import jax
import jax.numpy as jnp
from jax import lax
from jax.experimental import pallas as pl
from jax.experimental.pallas import tpu as pltpu
from jax.experimental.pallas import tpu_sc as plsc

B = 16384
H = B // 2
D = 64
W = 2 * D
NC, NS = 2, 16
NW = NC * NS
CHUNK = 128

_sc_mesh = plsc.VectorSubcoreMesh(core_axis_name="c", subcore_axis_name="s")


def _make_gather(nb):
    rows_w = nb // NW
    nchunk = rows_w // CHUNK

    @functools.partial(
        pl.kernel,
        out_type=jax.ShapeDtypeStruct((nb, W), jnp.float32),
        mesh=_sc_mesh,
        scratch_types=(
            pltpu.VMEM((nchunk, CHUNK), jnp.int32),
            pltpu.VMEM((nchunk, CHUNK, W), jnp.float32),
            tuple(pltpu.SemaphoreType.DMA for _ in range(2 * nchunk)),
        ),
    )
    def _gather(idx_hbm, tab, out, idx_v, bufs, sems):
        wid = lax.axis_index("s") * NC + lax.axis_index("c")
        base = wid * rows_w
        for j in range(nchunk):
            pltpu.sync_copy(idx_hbm.at[pl.ds(base + j * CHUNK, CHUNK)],
                            idx_v.at[j])
        copies = [
            pltpu.async_copy(tab.at[idx_v.at[j]], bufs.at[j], sems[j])
            for j in range(nchunk)
        ]
        wcopies = []
        for j in range(nchunk):
            copies[j].wait()
            wcopies.append(pltpu.async_copy(
                bufs.at[j], out.at[pl.ds(base + j * CHUNK, CHUNK)],
                sems[nchunk + j]))
        for c in wcopies:
            c.wait()

    return _gather


_gather_full = _make_gather(B)
_gather_quarter = _make_gather(B // 4)

_TDN = (((0,), (0,)), ((), ()))


def _concat_body(a_t, b_t, eye, out):
    e = eye[:]
    out[:, :D] = jax.lax.dot_general(
        a_t[:], e, _TDN, preferred_element_type=jnp.float32)
    out[:, D:] = jax.lax.dot_general(
        b_t[:], e, _TDN, preferred_element_type=jnp.float32)


_RC = 4096


def _concat_pair(a, b):
    n = a.shape[0]
    eye = jnp.eye(D, dtype=jnp.float32)
    return pl.pallas_call(
        _concat_body,
        grid=(pl.cdiv(n, _RC),),
        in_specs=[pl.BlockSpec((D, _RC), lambda i: (0, i)),
                  pl.BlockSpec((D, _RC), lambda i: (0, i)),
                  pl.BlockSpec((D, D), lambda i: (0, 0))],
        out_specs=pl.BlockSpec((_RC, W), lambda i: (i, 0)),
        out_shape=jax.ShapeDtypeStruct((n, W), jnp.float32),
    )(a.T, b.T, eye)


def _dense_body(u, i, w1u, w1i, b1, w2, b2, wpg, wph, bp, eye, out):
    uv = u[:]
    iv = i[:]
    g = uv[:, :D] * iv[:, :D]
    h1 = jnp.maximum(
        jnp.dot(uv[:, D:], w1u[:], preferred_element_type=jnp.float32)
        + jnp.dot(iv[:, D:], w1i[:], preferred_element_type=jnp.float32)
        + b1[:], 0.0)
    h2 = jnp.maximum(
        jnp.dot(h1, w2[:], preferred_element_type=jnp.float32) + b2[:], 0.0)
    col = (jnp.dot(g, wpg[:], preferred_element_type=jnp.float32)
           + jnp.dot(h2, wph[:], preferred_element_type=jnp.float32)
           + bp[0, 0])
    e = eye[:]
    for a in range(_R // 128):
        sub = col[a * 128:(a + 1) * 128, :]
        row = jax.lax.dot_general(sub, e, _TDN,
                                  preferred_element_type=jnp.float32)
        x = row.reshape(128)
        out[pl.ds(a * 128, 128)] = 1.0 / (1.0 + jnp.exp(-x))


_R = 2048


_Q = B // 4


def _dense_quarter(u_rows, i_q, weights, q):
    w1u, w1i, b1r, w2, b2r, wpg, wph, bpr, eye128 = weights
    off = q * (_Q // _R)
    full = lambda shape: pl.BlockSpec(shape, lambda i: (0, 0))
    return pl.pallas_call(
        _dense_body,
        grid=(_Q // _R,),
        in_specs=[
            pl.BlockSpec((_R, W), lambda i: (i + off, 0)),
            pl.BlockSpec((_R, W), lambda i: (i, 0)),
            full((D, 64)), full((D, 64)), full((1, 64)),
            full((64, 32)), full((1, 32)),
            full((D, 1)), full((32, 1)), full((1, 1)),
            full((128, 128)),
        ],
        out_specs=pl.BlockSpec((_R,), lambda i: (i,)),
        out_shape=jax.ShapeDtypeStruct((_Q,), jnp.float32),
    )(u_rows, i_q, w1u, w1i, b1r, w2, b2r, wpg, wph, bpr, eye128)


def kernel(user_indices, item_indices, embed_user_GMF, embed_item_GMF,
           embed_user_MLP, embed_item_MLP, W1, b1, W2, b2, Wp, bp):
    cat_u = _concat_pair(embed_user_GMF, embed_user_MLP)
    u_rows = _gather_full(user_indices, cat_u)
    cat_i = _concat_pair(embed_item_GMF, embed_item_MLP)
    i_q = [_gather_quarter(item_indices[k * _Q:(k + 1) * _Q], cat_i)
           for k in range(4)]

    weights = (W1[:D], W1[D:], b1.reshape(1, 64), W2, b2.reshape(1, 32),
               Wp[:D], Wp[D:], bp.reshape(1, 1),
               jnp.eye(128, dtype=jnp.float32))
    outs = [_dense_quarter(u_rows, i_q[k], weights, k) for k in range(4)]
    return jnp.concatenate(outs)


# speedup vs baseline: 1.1016x; 1.1016x over previous
import functools

import jax
import jax.numpy as jnp
from jax import lax
from jax.experimental import pallas as pl
from jax.experimental.pallas import tpu as pltpu
from jax.experimental.pallas import tpu_sc as plsc

B = 16384
H = B // 2
D = 64
W = 2 * D
NC, NS = 2, 16
NW = NC * NS
CHUNK = 128

_sc_mesh = plsc.VectorSubcoreMesh(core_axis_name="c", subcore_axis_name="s")


def _make_gather(nb):
    rows_w = nb // NW
    nchunk = rows_w // CHUNK

    @functools.partial(
        pl.kernel,
        out_type=jax.ShapeDtypeStruct((nb, W), jnp.float32),
        mesh=_sc_mesh,
        scratch_types=(
            pltpu.VMEM((nchunk, CHUNK), jnp.int32),
            pltpu.VMEM((nchunk, CHUNK, W), jnp.float32),
            tuple(pltpu.SemaphoreType.DMA for _ in range(2 * nchunk)),
        ),
    )
    def _gather(idx_hbm, tab, out, idx_v, bufs, sems):
        wid = lax.axis_index("s") * NC + lax.axis_index("c")
        base = wid * rows_w
        for j in range(nchunk):
            pltpu.sync_copy(idx_hbm.at[pl.ds(base + j * CHUNK, CHUNK)],
                            idx_v.at[j])
        copies = [
            pltpu.async_copy(tab.at[idx_v.at[j]], bufs.at[j], sems[j])
            for j in range(nchunk)
        ]
        wcopies = []
        for j in range(nchunk):
            copies[j].wait()
            wcopies.append(pltpu.async_copy(
                bufs.at[j], out.at[pl.ds(base + j * CHUNK, CHUNK)],
                sems[nchunk + j]))
        for c in wcopies:
            c.wait()

    return _gather


_gather_full = _make_gather(B)
_gather_half = _make_gather(H)

_TDN = (((0,), (0,)), ((), ()))


def _concat_body(a_t, b_t, eye, out):
    e = eye[:]
    out[:, :D] = jax.lax.dot_general(
        a_t[:], e, _TDN, preferred_element_type=jnp.float32)
    out[:, D:] = jax.lax.dot_general(
        b_t[:], e, _TDN, preferred_element_type=jnp.float32)


_RC = 4096


def _concat_pair(a, b):
    n = a.shape[0]
    eye = jnp.eye(D, dtype=jnp.float32)
    return pl.pallas_call(
        _concat_body,
        grid=(pl.cdiv(n, _RC),),
        in_specs=[pl.BlockSpec((D, _RC), lambda i: (0, i)),
                  pl.BlockSpec((D, _RC), lambda i: (0, i)),
                  pl.BlockSpec((D, D), lambda i: (0, 0))],
        out_specs=pl.BlockSpec((_RC, W), lambda i: (i, 0)),
        out_shape=jax.ShapeDtypeStruct((n, W), jnp.float32),
    )(a.T, b.T, eye)


def _dense_body(u, i, w1u, w1i, b1, w2, b2, wpg, wph, bp, eye, out):
    uv = u[:]
    iv = i[:]
    g = uv[:, :D] * iv[:, :D]
    h1 = jnp.maximum(
        jnp.dot(uv[:, D:], w1u[:], preferred_element_type=jnp.float32)
        + jnp.dot(iv[:, D:], w1i[:], preferred_element_type=jnp.float32)
        + b1[:], 0.0)
    h2 = jnp.maximum(
        jnp.dot(h1, w2[:], preferred_element_type=jnp.float32) + b2[:], 0.0)
    col = (jnp.dot(g, wpg[:], preferred_element_type=jnp.float32)
           + jnp.dot(h2, wph[:], preferred_element_type=jnp.float32)
           + bp[0, 0])
    e = eye[:]
    for a in range(_R // 128):
        sub = col[a * 128:(a + 1) * 128, :]
        row = jax.lax.dot_general(sub, e, _TDN,
                                  preferred_element_type=jnp.float32)
        x = row.reshape(128)
        out[pl.ds(a * 128, 128)] = 1.0 / (1.0 + jnp.exp(-x))


_R = 4096


def _dense_half(u_rows, i_half, weights, half):
    w1u, w1i, b1r, w2, b2r, wpg, wph, bpr, eye128 = weights
    off = half * (H // _R)
    full = lambda shape: pl.BlockSpec(shape, lambda i: (0, 0))
    return pl.pallas_call(
        _dense_body,
        grid=(H // _R,),
        in_specs=[
            pl.BlockSpec((_R, W), lambda i: (i + off, 0)),
            pl.BlockSpec((_R, W), lambda i: (i, 0)),
            full((D, 64)), full((D, 64)), full((1, 64)),
            full((64, 32)), full((1, 32)),
            full((D, 1)), full((32, 1)), full((1, 1)),
            full((128, 128)),
        ],
        out_specs=pl.BlockSpec((_R,), lambda i: (i,)),
        out_shape=jax.ShapeDtypeStruct((H,), jnp.float32),
    )(u_rows, i_half, w1u, w1i, b1r, w2, b2r, wpg, wph, bpr, eye128)


def kernel(user_indices, item_indices, embed_user_GMF, embed_item_GMF,
           embed_user_MLP, embed_item_MLP, W1, b1, W2, b2, Wp, bp):
    cat_u = _concat_pair(embed_user_GMF, embed_user_MLP)
    u_rows = _gather_full(user_indices, cat_u)
    cat_i = _concat_pair(embed_item_GMF, embed_item_MLP)
    i_h0 = _gather_half(item_indices[:H], cat_i)
    i_h1 = _gather_half(item_indices[H:], cat_i)

    weights = (W1[:D], W1[D:], b1.reshape(1, 64), W2, b2.reshape(1, 32),
               Wp[:D], Wp[D:], bp.reshape(1, 1),
               jnp.eye(128, dtype=jnp.float32))
    out0 = _dense_half(u_rows, i_h0, weights, 0)
    out1 = _dense_half(u_rows, i_h1, weights, 1)
    return jnp.concatenate([out0, out1])
